# Initial kernel scaffold; baseline (speedup 1.0000x reference)
import jax
import jax.numpy as jnp
from jax.experimental import pallas as pl


def kernel(x, embedding):
    raise NotImplementedError("write your pallas kernel here")


# speedup vs baseline: 1.8448x; 1.8448x over previous
import functools

import jax
import jax.numpy as jnp
from jax import lax
from jax.experimental import pallas as pl
from jax.experimental.pallas import tpu as pltpu
from jax.experimental.pallas import tpu_sc as plsc

VOCAB = 1000000
EMBED_DIM = 64
BATCH = 16384
HIST = 50

_B = BATCH * HIST
_NC, _NS = 2, 16
_NW = _NC * _NS
_BPW = _B // _NW
_CH = 128
_NK = 8
_SUP = _NK * _CH
_NSUP = _BPW // _SUP

_mesh = plsc.VectorSubcoreMesh(core_axis_name="c", subcore_axis_name="s")


@functools.partial(
    pl.kernel,
    out_type=jax.ShapeDtypeStruct((_B, EMBED_DIM), jnp.float32),
    mesh=_mesh,
    scratch_types=[
        pltpu.VMEM((_NK, _CH), jnp.int32),
        pltpu.VMEM((_SUP, EMBED_DIM), jnp.float32),
        pltpu.SemaphoreType.DMA,
    ],
    compiler_params=pltpu.CompilerParams(use_tc_tiling_on_sc=False),
)
def _gather_kernel(idx_hbm, table_hbm, out_hbm, idx_v, rows_v, sem):
    wid = lax.axis_index("s") * _NC + lax.axis_index("c")
    base = wid * _BPW

    def sup_body(i, carry):
        off = pl.multiple_of(base + i * _SUP, _SUP)
        row0 = pl.multiple_of(off // _CH, _NK)
        pltpu.sync_copy(idx_hbm.at[pl.ds(row0, _NK)], idx_v)
        copies = []
        for j in range(_NK):
            copies.append(
                pltpu.async_copy(
                    table_hbm.at[idx_v.at[j]],
                    rows_v.at[pl.ds(j * _CH, _CH)],
                    sem,
                )
            )
        for c in copies:
            c.wait()
        pltpu.sync_copy(rows_v, out_hbm.at[pl.ds(off, _SUP)])
        return carry

    lax.fori_loop(0, _NSUP, sup_body, 0)


def kernel(x, embedding):
    idx = x.reshape(_B // _CH, _CH).astype(jnp.int32)
    out = _gather_kernel(idx, embedding)
    return out.reshape(BATCH, HIST, EMBED_DIM)


# speedup vs baseline: 1.8746x; 1.0162x over previous
import functools

import jax
import jax.numpy as jnp
from jax import lax
from jax.experimental import pallas as pl
from jax.experimental.pallas import tpu as pltpu
from jax.experimental.pallas import tpu_sc as plsc

VOCAB = 1000000
EMBED_DIM = 64
BATCH = 16384
HIST = 50

_B = BATCH * HIST
_NC, _NS = 2, 16
_NW = _NC * _NS
_BPW = _B // _NW
_CH = 128
_NK = 4
_SUP = _NK * _CH
_NSUP = _BPW // _SUP
_NIR = _BPW // _CH

_mesh = plsc.VectorSubcoreMesh(core_axis_name="c", subcore_axis_name="s")


@functools.partial(
    pl.kernel,
    out_type=jax.ShapeDtypeStruct((_B, EMBED_DIM), jnp.float32),
    mesh=_mesh,
    scratch_types=[
        pltpu.VMEM((_NIR, _CH), jnp.int32),
        pltpu.VMEM((_SUP, EMBED_DIM), jnp.float32),
        pltpu.VMEM((_SUP, EMBED_DIM), jnp.float32),
        pltpu.SemaphoreType.DMA,
        pltpu.SemaphoreType.DMA,
        pltpu.SemaphoreType.DMA,
        pltpu.SemaphoreType.DMA,
    ],
    compiler_params=pltpu.CompilerParams(use_tc_tiling_on_sc=False),
)
def _gather_kernel(idx_hbm, table_hbm, out_hbm, idx_all, rows0, rows1,
                   sg0, sg1, ss0, ss1):
    wid = lax.axis_index("s") * _NC + lax.axis_index("c")
    base = wid * _BPW
    rows = (rows0, rows1)
    sg = (sg0, sg1)
    ss = (ss0, ss1)

    pltpu.sync_copy(
        idx_hbm.at[pl.ds(pl.multiple_of(base // _CH, 8), _NIR)], idx_all
    )

    def out_slice(g):
        return out_hbm.at[pl.ds(pl.multiple_of(base + g * _SUP, _SUP), _SUP)]

    def fire_gathers(g, b):
        for j in range(_NK):
            pltpu.async_copy(
                table_hbm.at[idx_all.at[g * _NK + j]],
                rows[b].at[pl.ds(j * _CH, _CH)],
                sg[b],
            )

    def drain_gathers(b):
        pltpu.make_async_copy(out_slice(0), rows[b], sg[b]).wait()

    def wait_store(g, b):
        pltpu.make_async_copy(rows[b], out_slice(g), ss[b]).wait()

    fire_gathers(0, 0)
    fire_gathers(1, 1)
    drain_gathers(0)
    pltpu.async_copy(rows[0], out_slice(0), ss[0])

    def outer(i, carry):
        for b in range(2):
            g = 2 * i + b
            wait_store(g - 2, b)
            fire_gathers(g, b)
            drain_gathers(1 - b)
            pltpu.async_copy(rows[1 - b], out_slice(g - 1), ss[1 - b])
        return carry

    lax.fori_loop(1, _NSUP // 2, outer, 0)

    wait_store(_NSUP - 2, 0)
    drain_gathers(1)
    pltpu.async_copy(rows[1], out_slice(_NSUP - 1), ss[1])
    wait_store(_NSUP - 1, 1)


def kernel(x, embedding):
    idx = x.reshape(_B // _CH, _CH).astype(jnp.int32)
    out = _gather_kernel(idx, embedding)
    return out.reshape(BATCH, HIST, EMBED_DIM)
